# baseline (speedup 1.0000x reference)
import jax
import jax.numpy as jnp
from jax.experimental import pallas as pl
from jax.experimental.pallas import tpu as pltpu

_B = 1024
_D_IN = 64
_H = 1024
_D_OUT = 64
_K = 1024
_EDIM = 64
_N = _B * _D_IN
_T = 8
_W = 1024 * _T
_STEPS = _EDIM // _T


def _nt_dot(a, b):
    return jax.lax.dot_general(a, b, (((1,), (1,)), ((), ())),
                               preferred_element_type=jnp.float32)


def _fused_kernel(xr_ref, emb_ref, x_ref, w1_ref, b1_ref, wh_ref, bh_ref,
                  wo_ref, bo_ref, f_ref, loss_ref, ind_ref, qlat_ref):
    j = pl.program_id(0)

    @pl.when(j < _STEPS)
    def _dist_step():
        xb = xr_ref[0].reshape(1, _W)
        xbb = jnp.broadcast_to(xb, (_EDIM, _W))
        e = jax.lax.broadcasted_iota(jnp.int32, (_EDIM, _W), 0) + 1
        pw = xbb
        acc = jnp.ones((_EDIM, _W), jnp.float32)
        for b in range(7):
            acc = jnp.where(((e >> b) & 1) == 1, acc * pw, acc)
            if b < 6:
                pw = pw * pw
        P = acc
        emb = emb_ref[...]
        embsq = jnp.sum(emb * emb, axis=1, keepdims=True)
        sm = jnp.sum(P, axis=0, keepdims=True)
        dist = (sm + embsq) + jnp.dot(
            emb * -2.0, P, preferred_element_type=jnp.float32)
        h = _K // 2
        mask = dist[:h] <= dist[h:]
        vals = jnp.minimum(dist[:h], dist[h:])
        idxs = jnp.where(mask, jnp.int32(0), jnp.int32(h))
        h //= 2
        while h >= 8:
            mask = vals[:h] <= vals[h:]
            vals = jnp.minimum(vals[:h], vals[h:])
            idxs = jnp.where(mask, idxs[:h], idxs[h:] + jnp.int32(h))
            h //= 2
        idxs = idxs + jax.lax.broadcasted_iota(jnp.int32, (8, _W), 0)
        minv = jnp.min(vals, axis=0)
        amin = jnp.min(
            jnp.where(vals == minv[None, :], idxs, jnp.int32(1 << 30)),
            axis=0)
        ind_ref[pl.ds(j, 1)] = amin.astype(jnp.float32).reshape(1, _T, 1024)
        rowsq = jnp.sum(P * P, axis=0)
        part = jnp.sum(rowsq + (minv - sm[0]))

        @pl.when(j == 0)
        def _init():
            qlat_ref[0, 0] = 0.0

        qlat_ref[0, 0] += part

    @pl.when(j == _STEPS)
    def _mlp_step():
        q = jnp.transpose(ind_ref[...].reshape(_EDIM, 1024))
        h1 = jnp.maximum(_nt_dot(q, w1_ref[...]) + b1_ref[...], 0.0)
        for _ in range(4):
            h1 = jnp.maximum(_nt_dot(h1, wh_ref[...]) + bh_ref[...], 0.0)
        f_ref[...] = jnp.maximum(_nt_dot(h1, wo_ref[...]) + bo_ref[...], 0.0)
        d = x_ref[...] - q
        e_sum = jnp.sum(d * d)
        loss_ref[...] = (qlat_ref[0, 0] * (1.0 / (_N * _EDIM))
                         + 0.25 * e_sum * (1.0 / _N)).reshape(1, 1)


def kernel(x, emb_w, W1, b1, Wh, bh, Wo, bo):
    xr3 = x.reshape(_STEPS, _T, 1024)
    last = _STEPS - 1

    const = lambda *blk: pl.BlockSpec(blk, lambda j: tuple(0 for _ in blk))
    f, loss = pl.pallas_call(
        _fused_kernel,
        grid=(_STEPS + 1,),
        in_specs=[
            pl.BlockSpec((1, _T, 1024), lambda j: (jnp.minimum(j, last), 0, 0)),
            const(_K, _EDIM),
            const(_B, _D_IN),
            const(_H, _D_IN),
            const(1, _H),
            const(_H, _H),
            const(1, _H),
            const(_D_OUT, _H),
            const(1, _D_OUT),
        ],
        out_specs=[
            const(_B, _D_OUT),
            const(1, 1),
        ],
        out_shape=[
            jax.ShapeDtypeStruct((_B, _D_OUT), jnp.float32),
            jax.ShapeDtypeStruct((1, 1), jnp.float32),
        ],
        scratch_shapes=[
            pltpu.VMEM((_STEPS, _T, 1024), jnp.float32),
            pltpu.SMEM((1, 1), jnp.float32),
        ],
    )(xr3, emb_w, x, W1, b1.reshape(1, _H), Wh, bh.reshape(1, _H),
      Wo, bo.reshape(1, _D_OUT))

    return f, loss[0, 0]


# speedup vs baseline: 1.0743x; 1.0743x over previous
import jax
import jax.numpy as jnp
from jax.experimental import pallas as pl
from jax.experimental.pallas import tpu as pltpu

_B = 1024
_D_IN = 64
_H = 1024
_D_OUT = 64
_K = 1024
_EDIM = 64
_N = _B * _D_IN
_T = 4
_W = 1024 * _T
_STEPS = _EDIM // _T


def _nt_dot(a, b):
    return jax.lax.dot_general(a, b, (((1,), (1,)), ((), ())),
                               preferred_element_type=jnp.float32)


def _fused_kernel(xr_ref, emb_ref, x_ref, w1_ref, b1_ref, wh_ref, bh_ref,
                  wo_ref, bo_ref, f_ref, loss_ref, ind_ref, qlat_ref):
    j = pl.program_id(0)

    @pl.when(j < _STEPS)
    def _dist_step():
        xb = xr_ref[0].reshape(1, _W)
        e8 = jax.lax.broadcasted_iota(jnp.int32, (8, _W), 0) + 1
        pw = jnp.broadcast_to(xb, (8, _W))
        acc = jnp.ones((8, _W), jnp.float32)
        for b in range(4):
            acc = jnp.where(((e8 >> b) & 1) == 1, acc * pw, acc)
            if b < 3:
                pw = pw * pw
        P = acc
        for t in (8, 16, 32):
            P = jnp.concatenate([P, P * P[t - 1:t]], axis=0)
        emb = emb_ref[...]
        embsq = jnp.sum(emb * emb, axis=1, keepdims=True)
        sm = jnp.sum(P, axis=0, keepdims=True)
        dist = (sm + embsq) + jnp.dot(
            emb * -2.0, P, preferred_element_type=jnp.float32)
        h = _K // 2
        mask = dist[:h] <= dist[h:]
        vals = jnp.minimum(dist[:h], dist[h:])
        idxs = jnp.where(mask, jnp.int32(0), jnp.int32(h))
        h //= 2
        while h >= 8:
            mask = vals[:h] <= vals[h:]
            vals = jnp.minimum(vals[:h], vals[h:])
            idxs = jnp.where(mask, idxs[:h], idxs[h:] + jnp.int32(h))
            h //= 2
        idxs = idxs + jax.lax.broadcasted_iota(jnp.int32, (8, _W), 0)
        minv = jnp.min(vals, axis=0)
        amin = jnp.min(
            jnp.where(vals == minv[None, :], idxs, jnp.int32(1 << 30)),
            axis=0)
        ind_ref[pl.ds(j, 1)] = amin.astype(jnp.float32).reshape(1, _T, 1024)
        rowsq = jnp.sum(P * P, axis=0)
        part = jnp.sum(rowsq + (minv - sm[0]))

        @pl.when(j == 0)
        def _init():
            qlat_ref[0, 0] = 0.0

        qlat_ref[0, 0] += part

    @pl.when(j == _STEPS)
    def _mlp_step():
        q = jnp.transpose(ind_ref[...].reshape(_EDIM, 1024))
        h1 = jnp.maximum(_nt_dot(q, w1_ref[...]) + b1_ref[...], 0.0)
        for _ in range(4):
            h1 = jnp.maximum(_nt_dot(h1, wh_ref[...]) + bh_ref[...], 0.0)
        f_ref[...] = jnp.maximum(_nt_dot(h1, wo_ref[...]) + bo_ref[...], 0.0)
        d = x_ref[...] - q
        e_sum = jnp.sum(d * d)
        loss_ref[...] = (qlat_ref[0, 0] * (1.0 / (_N * _EDIM))
                         + 0.25 * e_sum * (1.0 / _N)).reshape(1, 1)


def kernel(x, emb_w, W1, b1, Wh, bh, Wo, bo):
    xr3 = x.reshape(_STEPS, _T, 1024)
    last = _STEPS - 1

    const = lambda *blk: pl.BlockSpec(blk, lambda j: tuple(0 for _ in blk))
    f, loss = pl.pallas_call(
        _fused_kernel,
        grid=(_STEPS + 1,),
        in_specs=[
            pl.BlockSpec((1, _T, 1024), lambda j: (jnp.minimum(j, last), 0, 0)),
            const(_K, _EDIM),
            const(_B, _D_IN),
            const(_H, _D_IN),
            const(1, _H),
            const(_H, _H),
            const(1, _H),
            const(_D_OUT, _H),
            const(1, _D_OUT),
        ],
        out_specs=[
            const(_B, _D_OUT),
            const(1, 1),
        ],
        out_shape=[
            jax.ShapeDtypeStruct((_B, _D_OUT), jnp.float32),
            jax.ShapeDtypeStruct((1, 1), jnp.float32),
        ],
        scratch_shapes=[
            pltpu.VMEM((_STEPS, _T, 1024), jnp.float32),
            pltpu.SMEM((1, 1), jnp.float32),
        ],
    )(xr3, emb_w, x, W1, b1.reshape(1, _H), Wh, bh.reshape(1, _H),
      Wo, bo.reshape(1, _D_OUT))

    return f, loss[0, 0]


# speedup vs baseline: 1.0810x; 1.0063x over previous
import jax
import jax.numpy as jnp
from jax.experimental import pallas as pl
from jax.experimental.pallas import tpu as pltpu

_B = 1024
_D_IN = 64
_H = 1024
_D_OUT = 64
_K = 1024
_EDIM = 64
_N = _B * _D_IN
_T = 4
_W = 1024 * _T
_STEPS = _EDIM // _T


def _nt_dot(a, b):
    return jax.lax.dot_general(a, b, (((1,), (1,)), ((), ())),
                               preferred_element_type=jnp.float32)


def _fused_kernel(xr_ref, emb_ref, x_ref, w1_ref, b1_ref, wh_ref, bh_ref,
                  wo_ref, bo_ref, f_ref, loss_ref, ind_ref, qlat_ref):
    j = pl.program_id(0)

    @pl.when(j < _STEPS)
    def _dist_step():
        xb = xr_ref[0].reshape(1, _W)
        e8 = jax.lax.broadcasted_iota(jnp.int32, (8, _W), 0) + 1
        pw = jnp.broadcast_to(xb, (8, _W))
        acc = jnp.ones((8, _W), jnp.float32)
        for b in range(4):
            acc = jnp.where(((e8 >> b) & 1) == 1, acc * pw, acc)
            if b < 3:
                pw = pw * pw
        P = acc
        for t in (8, 16, 32):
            P = jnp.concatenate([P, P * P[t - 1:t]], axis=0)
        emb = emb_ref[...]
        embsq = jnp.sum(emb * emb, axis=1, keepdims=True)
        sm = jnp.sum(P, axis=0, keepdims=True)
        dist = (sm + embsq) + jnp.dot(
            emb * -2.0, P, preferred_element_type=jnp.float32)
        h = _K // 2
        mask = dist[:h] <= dist[h:]
        vals = jnp.minimum(dist[:h], dist[h:])
        idxs = jnp.where(mask, jnp.int32(0), jnp.int32(h))
        h //= 2
        while h >= 8:
            mask = vals[:h] <= vals[h:]
            vals = jnp.minimum(vals[:h], vals[h:])
            idxs = jnp.where(mask, idxs[:h], idxs[h:] + jnp.int32(h))
            h //= 2
        idxs = idxs + jax.lax.broadcasted_iota(jnp.int32, (8, _W), 0)
        minv = jnp.min(vals, axis=0)
        amin = jnp.min(
            jnp.where(vals == minv[None, :], idxs, jnp.int32(1 << 30)),
            axis=0)
        ind_ref[pl.ds(j, 1)] = amin.astype(jnp.float32).reshape(1, _T, 1024)
        a = xb * xb
        den = 1.0 - a
        x64 = P[_EDIM - 1:_EDIM]
        rowsq = jnp.where(den > 0.0, a * (1.0 - x64 * x64) / den, 64.0)
        part = jnp.sum(rowsq[0] + (minv - sm[0]))

        @pl.when(j == 0)
        def _init():
            qlat_ref[0, 0] = 0.0

        qlat_ref[0, 0] += part

    @pl.when(j == _STEPS)
    def _mlp_step():
        q = jnp.transpose(ind_ref[...].reshape(_EDIM, 1024))
        h1 = jnp.maximum(_nt_dot(q, w1_ref[...]) + b1_ref[...], 0.0)
        for _ in range(4):
            h1 = jnp.maximum(_nt_dot(h1, wh_ref[...]) + bh_ref[...], 0.0)
        f_ref[...] = jnp.maximum(_nt_dot(h1, wo_ref[...]) + bo_ref[...], 0.0)
        d = x_ref[...] - q
        e_sum = jnp.sum(d * d)
        loss_ref[...] = (qlat_ref[0, 0] * (1.0 / (_N * _EDIM))
                         + 0.25 * e_sum * (1.0 / _N)).reshape(1, 1)


def kernel(x, emb_w, W1, b1, Wh, bh, Wo, bo):
    xr3 = x.reshape(_STEPS, _T, 1024)
    last = _STEPS - 1

    const = lambda *blk: pl.BlockSpec(blk, lambda j: tuple(0 for _ in blk))
    f, loss = pl.pallas_call(
        _fused_kernel,
        grid=(_STEPS + 1,),
        in_specs=[
            pl.BlockSpec((1, _T, 1024), lambda j: (jnp.minimum(j, last), 0, 0)),
            const(_K, _EDIM),
            const(_B, _D_IN),
            const(_H, _D_IN),
            const(1, _H),
            const(_H, _H),
            const(1, _H),
            const(_D_OUT, _H),
            const(1, _D_OUT),
        ],
        out_specs=[
            const(_B, _D_OUT),
            const(1, 1),
        ],
        out_shape=[
            jax.ShapeDtypeStruct((_B, _D_OUT), jnp.float32),
            jax.ShapeDtypeStruct((1, 1), jnp.float32),
        ],
        scratch_shapes=[
            pltpu.VMEM((_STEPS, _T, 1024), jnp.float32),
            pltpu.SMEM((1, 1), jnp.float32),
        ],
    )(xr3, emb_w, x, W1, b1.reshape(1, _H), Wh, bh.reshape(1, _H),
      Wo, bo.reshape(1, _D_OUT))

    return f, loss[0, 0]
